# baseline (speedup 1.0000x reference)
import functools
import math

import jax
import jax.numpy as jnp
from jax import lax
from jax.experimental import pallas as pl
from jax.experimental.pallas import tpu as pltpu
from jax.experimental.pallas import tpu_sc as plsc

_THRESH = 0.7
_C07 = math.log(_THRESH / (1.0 - _THRESH))
_RADIUS = 7

_INTERPRET = False


def _per_pixel(score_ref, target_ref):
    t = target_ref[0]
    s0 = score_ref[0, 0]
    s1 = score_ref[0, 1]
    tb = t == 1
    d = jnp.where(tb, s1 - s0, s0 - s1)
    l = jnp.where(tb, -s1, -s0)
    return t, tb, d, l


def _pack_row_scalars(vals):
    row = lax.broadcasted_iota(jnp.int32, (8, 128), 0)
    acc = jnp.zeros((8, 128), jnp.float32)
    for i, v in enumerate(vals):
        acc = jnp.where(row == i, v, acc)
    return acc


def _main_body(score_ref, target_ref, out_ref):
    b = pl.program_id(0)
    t, _, d, l = _per_pixel(score_ref, target_ref)
    H, W = t.shape

    r = lax.broadcasted_iota(jnp.int32, (H, H), 0)
    c = lax.broadcasted_iota(jnp.int32, (H, H), 1)
    band_h = (jnp.abs(r - c) <= _RADIUS).astype(jnp.bfloat16)
    rw = lax.broadcasted_iota(jnp.int32, (W, W), 0)
    cw = lax.broadcasted_iota(jnp.int32, (W, W), 1)
    band_w = (jnp.abs(rw - cw) <= _RADIUS).astype(jnp.bfloat16)

    tf = t.astype(jnp.bfloat16)
    srow = lax.dot_general(band_h, tf, (((1,), (0,)), ((), ())),
                           preferred_element_type=jnp.float32)
    sbox = lax.dot_general(srow.astype(jnp.bfloat16), band_w,
                           (((1,), (0,)), ((), ())),
                           preferred_element_type=jnp.float32)

    ri = lax.broadcasted_iota(jnp.int32, (H, W), 0)
    ci = lax.broadcasted_iota(jnp.int32, (H, W), 1)
    cnt_r = jnp.minimum(ri, _RADIUS) + jnp.minimum(H - 1 - ri, _RADIUS) + 1
    cnt_c = jnp.minimum(ci, _RADIUS) + jnp.minimum(W - 1 - ci, _RADIUS) + 1
    nwin = (cnt_r * cnt_c).astype(jnp.float32)
    boundary = (sbox > 0.0) & (sbox < nwin)

    lt = d < _C07
    le = d <= _C07
    sum_lt = jnp.sum(jnp.where(lt, l, 0.0))
    cnt_lt = jnp.sum(lt.astype(jnp.float32))
    cnt_le = jnp.sum(le.astype(jnp.float32))
    edge_sum = jnp.sum(jnp.where(boundary, l, 0.0))
    edge_cnt = jnp.sum(boundary.astype(jnp.float32))

    acc = _pack_row_scalars([sum_lt, cnt_lt, cnt_le, edge_sum, edge_cnt])

    @pl.when(b == 0)
    def _():
        out_ref[...] = acc

    @pl.when(b != 0)
    def _():
        out_ref[...] = out_ref[...] + acc


def _edge_body(score_ref, target_ref, out_ref):
    b = pl.program_id(0)
    t, _, _, l = _per_pixel(score_ref, target_ref)
    H, W = t.shape

    r = lax.broadcasted_iota(jnp.int32, (H, H), 0)
    c = lax.broadcasted_iota(jnp.int32, (H, H), 1)
    band_h = (jnp.abs(r - c) <= _RADIUS).astype(jnp.bfloat16)
    rw = lax.broadcasted_iota(jnp.int32, (W, W), 0)
    cw = lax.broadcasted_iota(jnp.int32, (W, W), 1)
    band_w = (jnp.abs(rw - cw) <= _RADIUS).astype(jnp.bfloat16)

    tf = t.astype(jnp.bfloat16)
    srow = lax.dot_general(band_h, tf, (((1,), (0,)), ((), ())),
                           preferred_element_type=jnp.float32)
    sbox = lax.dot_general(srow.astype(jnp.bfloat16), band_w,
                           (((1,), (0,)), ((), ())),
                           preferred_element_type=jnp.float32)

    ri = lax.broadcasted_iota(jnp.int32, (H, W), 0)
    ci = lax.broadcasted_iota(jnp.int32, (H, W), 1)
    cnt_r = jnp.minimum(ri, _RADIUS) + jnp.minimum(H - 1 - ri, _RADIUS) + 1
    cnt_c = jnp.minimum(ci, _RADIUS) + jnp.minimum(W - 1 - ci, _RADIUS) + 1
    nwin = (cnt_r * cnt_c).astype(jnp.float32)
    boundary = (sbox > 0.0) & (sbox < nwin)

    edge_sum = jnp.sum(jnp.where(boundary, l, 0.0))
    edge_cnt = jnp.sum(boundary.astype(jnp.float32))
    acc = _pack_row_scalars([edge_sum, edge_cnt])

    @pl.when(b == 0)
    def _():
        out_ref[...] = acc

    @pl.when(b != 0)
    def _():
        out_ref[...] = out_ref[...] + acc


_SC_CHUNK = 16384


def _sc_ohem_partials(score3, target2):
    B, _, HW = score3.shape
    mesh = plsc.VectorSubcoreMesh(core_axis_name="c", subcore_axis_name="s")
    nw = mesh.num_cores * mesh.num_subcores
    px_per_w = (B * HW) // nw
    nch = px_per_w // _SC_CHUNK

    @functools.partial(
        pl.kernel,
        out_type=jax.ShapeDtypeStruct((nw, 64), jnp.float32),
        mesh=mesh,
        scratch_types=[
            pltpu.VMEM((_SC_CHUNK,), jnp.float32),
            pltpu.VMEM((_SC_CHUNK,), jnp.float32),
            pltpu.VMEM((_SC_CHUNK,), jnp.int32),
            pltpu.VMEM((64,), jnp.float32),
        ],
        interpret=_INTERPRET,
    )
    def k(score_hbm, target_hbm, out_hbm, s0_v, s1_v, t_v, res_v):
        cid = lax.axis_index("c")
        sid = lax.axis_index("s")
        wid = sid * mesh.num_cores + cid

        def chunk_body(j, accs):
            p0 = wid * px_per_w + j * _SC_CHUNK
            b = p0 // HW
            off = p0 % HW
            pltpu.sync_copy(score_hbm.at[b, 0, pl.ds(off, _SC_CHUNK)], s0_v)
            pltpu.sync_copy(score_hbm.at[b, 1, pl.ds(off, _SC_CHUNK)], s1_v)
            pltpu.sync_copy(target_hbm.at[b, pl.ds(off, _SC_CHUNK)], t_v)

            def vec_body(i, a):
                asum, aclt, acle = a
                sl = pl.ds(i * 16, 16)
                t = t_v[sl]
                s0 = s0_v[sl]
                s1 = s1_v[sl]
                tb = t == 1
                d = jnp.where(tb, s1 - s0, s0 - s1)
                l = jnp.where(tb, -s1, -s0)
                lt = d < _C07
                asum = asum + jnp.where(lt, l, 0.0)
                aclt = aclt + jnp.where(lt, 1.0, 0.0)
                acle = acle + jnp.where(d <= _C07, 1.0, 0.0)
                return (asum, aclt, acle)

            return lax.fori_loop(0, _SC_CHUNK // 16, vec_body, accs)

        z = jnp.zeros((16,), jnp.float32)
        asum, aclt, acle = lax.fori_loop(0, nch, chunk_body, (z, z, z))
        res_v[pl.ds(0, 16)] = asum
        res_v[pl.ds(16, 16)] = aclt
        res_v[pl.ds(32, 16)] = acle
        res_v[pl.ds(48, 16)] = z
        pltpu.sync_copy(res_v, out_hbm.at[wid])

    return k(score3, target2)


def _ikey(d):
    bits = lax.bitcast_convert_type(d, jnp.int32)
    return jnp.where(bits >= 0, bits,
                     jnp.bitwise_xor(jnp.bitwise_not(bits), jnp.int32(-(2 ** 31))))


def _rare_body(kplus1, nb, score_ref, target_ref, out_ref, st_ref):
    i = pl.program_id(0)
    b = pl.program_id(1)

    @pl.when((i == 0) & (b == 0))
    def _():
        st_ref[0] = jnp.int32(-(2 ** 31))
        st_ref[1] = jnp.int32(2 ** 31 - 1)

    @pl.when((i < 32) & (b == 0))
    def _():
        lo = st_ref[0]
        hi = st_ref[1]
        st_ref[2] = (lo >> 1) + (hi >> 1) + (lo & hi & 1)
        st_ref[3] = jnp.int32(0)

    _, tb, d, l = _per_pixel(score_ref, target_ref)
    key = _ikey(d)

    @pl.when(i < 32)
    def _():
        mid = st_ref[2]
        st_ref[3] = st_ref[3] + jnp.sum((key <= mid).astype(jnp.int32))

    @pl.when((i < 32) & (b == nb - 1))
    def _():
        take_hi = st_ref[3] >= kplus1
        lo = st_ref[0]
        hi = st_ref[1]
        mid = st_ref[2]
        st_ref[0] = jnp.where(take_hi, lo, mid + 1)
        st_ref[1] = jnp.where(take_hi, mid, hi)

    @pl.when(i == 32)
    def _():
        kstar = st_ref[0]
        keep = key < kstar
        ssum = jnp.sum(jnp.where(keep, l, 0.0))
        scnt = jnp.sum(keep.astype(jnp.float32))
        acc = _pack_row_scalars([ssum, scnt])

        @pl.when(b == 0)
        def _():
            out_ref[...] = acc

        @pl.when(b != 0)
        def _():
            out_ref[...] = out_ref[...] + acc


def _rare_ohem(score, target, kplus1):
    B, _, H, W = score.shape
    out = pl.pallas_call(
        lambda sr, tr, orf, st: _rare_body(kplus1, B, sr, tr, orf, st),
        grid=(33, B),
        in_specs=[
            pl.BlockSpec((1, 2, H, W), lambda i, b: (b, 0, 0, 0)),
            pl.BlockSpec((1, H, W), lambda i, b: (b, 0, 0)),
        ],
        out_specs=pl.BlockSpec((8, 128), lambda i, b: (0, 0)),
        out_shape=jax.ShapeDtypeStruct((8, 128), jnp.float32),
        scratch_shapes=[pltpu.SMEM((4,), jnp.int32)],
        compiler_params=pltpu.CompilerParams(
            dimension_semantics=("arbitrary", "arbitrary")),
        interpret=_INTERPRET,
    )(score, target)
    ssum = out[0, 0]
    scnt = out[1, 0]
    return ssum / jnp.maximum(scnt, 1.0)


def kernel(score, target):
    B, C, H, W = score.shape
    target = target.astype(jnp.int32)
    min_kept = int(0.7 * H * W)
    k = min(min_kept, B * H * W - 1)

    sc_out = _sc_ohem_partials(score.reshape(B, C, H * W),
                               target.reshape(B, H * W))
    out = pl.pallas_call(
        _edge_body,
        grid=(B,),
        in_specs=[
            pl.BlockSpec((1, C, H, W), lambda b: (b, 0, 0, 0)),
            pl.BlockSpec((1, H, W), lambda b: (b, 0, 0)),
        ],
        out_specs=pl.BlockSpec((8, 128), lambda b: (0, 0)),
        out_shape=jax.ShapeDtypeStruct((8, 128), jnp.float32),
        compiler_params=pltpu.CompilerParams(
            dimension_semantics=("arbitrary",)),
        interpret=_INTERPRET,
    )(score, target)

    sum_lt = jnp.sum(sc_out[:, 0:16])
    cnt_lt = jnp.sum(sc_out[:, 16:32])
    cnt_le = jnp.sum(sc_out[:, 32:48])
    edge_sum = out[0, 0]
    edge_cnt = out[1, 0]

    common = cnt_le >= jnp.float32(k + 1)
    ohem = lax.cond(
        common,
        lambda: sum_lt / jnp.maximum(cnt_lt, 1.0),
        lambda: _rare_ohem(score, target, k + 1),
    )
    edge = edge_sum / jnp.maximum(edge_cnt, 1.0)
    return ohem + 0.5 * edge


# speedup vs baseline: 4.5191x; 4.5191x over previous
import functools
import math

import jax
import jax.numpy as jnp
from jax import lax
from jax.experimental import pallas as pl
from jax.experimental.pallas import tpu as pltpu
from jax.experimental.pallas import tpu_sc as plsc

_THRESH = 0.7
_C07 = math.log(_THRESH / (1.0 - _THRESH))
_RADIUS = 7

_INTERPRET = False


def _per_pixel(score_ref, target_ref):
    t = target_ref[0]
    s0 = score_ref[0, 0]
    s1 = score_ref[0, 1]
    tb = t == 1
    d = jnp.where(tb, s1 - s0, s0 - s1)
    l = jnp.where(tb, -s1, -s0)
    return t, tb, d, l


def _pack_row_scalars(vals):
    row = lax.broadcasted_iota(jnp.int32, (8, 128), 0)
    acc = jnp.zeros((8, 128), jnp.float32)
    for i, v in enumerate(vals):
        acc = jnp.where(row == i, v, acc)
    return acc


def _build_consts(H, W, band_ref, nwin_ref):
    r = lax.broadcasted_iota(jnp.int32, (H, H), 0)
    c = lax.broadcasted_iota(jnp.int32, (H, H), 1)
    band_ref[...] = (jnp.abs(r - c) <= _RADIUS).astype(jnp.bfloat16)
    ri = lax.broadcasted_iota(jnp.int32, (H, W), 0)
    ci = lax.broadcasted_iota(jnp.int32, (H, W), 1)
    cnt_r = jnp.minimum(ri, _RADIUS) + jnp.minimum(H - 1 - ri, _RADIUS) + 1
    cnt_c = jnp.minimum(ci, _RADIUS) + jnp.minimum(W - 1 - ci, _RADIUS) + 1
    nwin_ref[...] = (cnt_r * cnt_c).astype(jnp.float32)


def _main_body(score_ref, target_ref, out_ref, band_ref, nwin_ref):
    b = pl.program_id(0)
    t, _, d, l = _per_pixel(score_ref, target_ref)
    H, W = t.shape

    @pl.when(b == 0)
    def _():
        _build_consts(H, W, band_ref, nwin_ref)

    band = band_ref[...]
    tf = t.astype(jnp.bfloat16)
    srow = lax.dot_general(band, tf, (((1,), (0,)), ((), ())),
                           preferred_element_type=jnp.float32)
    sbox = lax.dot_general(srow.astype(jnp.bfloat16), band,
                           (((1,), (0,)), ((), ())),
                           preferred_element_type=jnp.float32)
    boundary = (sbox > 0.0) & (sbox < nwin_ref[...])

    lt = d < _C07
    sum_lt = jnp.sum(jnp.where(lt, l, 0.0))
    cnt_lt = jnp.sum(lt.astype(jnp.float32))
    edge_sum = jnp.sum(jnp.where(boundary, l, 0.0))
    edge_cnt = jnp.sum(boundary.astype(jnp.float32))

    acc = _pack_row_scalars([sum_lt, cnt_lt, edge_sum, edge_cnt])

    @pl.when(b == 0)
    def _():
        out_ref[...] = acc

    @pl.when(b != 0)
    def _():
        out_ref[...] = out_ref[...] + acc


def _edge_body(score_ref, target_ref, out_ref):
    b = pl.program_id(0)
    t, _, _, l = _per_pixel(score_ref, target_ref)
    H, W = t.shape

    r = lax.broadcasted_iota(jnp.int32, (H, H), 0)
    c = lax.broadcasted_iota(jnp.int32, (H, H), 1)
    band_h = (jnp.abs(r - c) <= _RADIUS).astype(jnp.bfloat16)
    rw = lax.broadcasted_iota(jnp.int32, (W, W), 0)
    cw = lax.broadcasted_iota(jnp.int32, (W, W), 1)
    band_w = (jnp.abs(rw - cw) <= _RADIUS).astype(jnp.bfloat16)

    tf = t.astype(jnp.bfloat16)
    srow = lax.dot_general(band_h, tf, (((1,), (0,)), ((), ())),
                           preferred_element_type=jnp.float32)
    sbox = lax.dot_general(srow.astype(jnp.bfloat16), band_w,
                           (((1,), (0,)), ((), ())),
                           preferred_element_type=jnp.float32)

    ri = lax.broadcasted_iota(jnp.int32, (H, W), 0)
    ci = lax.broadcasted_iota(jnp.int32, (H, W), 1)
    cnt_r = jnp.minimum(ri, _RADIUS) + jnp.minimum(H - 1 - ri, _RADIUS) + 1
    cnt_c = jnp.minimum(ci, _RADIUS) + jnp.minimum(W - 1 - ci, _RADIUS) + 1
    nwin = (cnt_r * cnt_c).astype(jnp.float32)
    boundary = (sbox > 0.0) & (sbox < nwin)

    edge_sum = jnp.sum(jnp.where(boundary, l, 0.0))
    edge_cnt = jnp.sum(boundary.astype(jnp.float32))
    acc = _pack_row_scalars([edge_sum, edge_cnt])

    @pl.when(b == 0)
    def _():
        out_ref[...] = acc

    @pl.when(b != 0)
    def _():
        out_ref[...] = out_ref[...] + acc


_SC_CHUNK = 16384


def _sc_ohem_partials(score3, target2):
    B, _, HW = score3.shape
    mesh = plsc.VectorSubcoreMesh(core_axis_name="c", subcore_axis_name="s")
    nw = mesh.num_cores * mesh.num_subcores
    px_per_w = (B * HW) // nw
    nch = px_per_w // _SC_CHUNK

    @functools.partial(
        pl.kernel,
        out_type=jax.ShapeDtypeStruct((nw, 64), jnp.float32),
        mesh=mesh,
        scratch_types=[
            pltpu.VMEM((_SC_CHUNK,), jnp.float32),
            pltpu.VMEM((_SC_CHUNK,), jnp.float32),
            pltpu.VMEM((_SC_CHUNK,), jnp.int32),
            pltpu.VMEM((64,), jnp.float32),
        ],
        interpret=_INTERPRET,
    )
    def k(score_hbm, target_hbm, out_hbm, s0_v, s1_v, t_v, res_v):
        cid = lax.axis_index("c")
        sid = lax.axis_index("s")
        wid = sid * mesh.num_cores + cid

        def chunk_body(j, accs):
            p0 = wid * px_per_w + j * _SC_CHUNK
            b = p0 // HW
            off = p0 % HW
            pltpu.sync_copy(score_hbm.at[b, 0, pl.ds(off, _SC_CHUNK)], s0_v)
            pltpu.sync_copy(score_hbm.at[b, 1, pl.ds(off, _SC_CHUNK)], s1_v)
            pltpu.sync_copy(target_hbm.at[b, pl.ds(off, _SC_CHUNK)], t_v)

            def vec_body(i, a):
                asum, aclt, acle = a
                sl = pl.ds(i * 16, 16)
                t = t_v[sl]
                s0 = s0_v[sl]
                s1 = s1_v[sl]
                tb = t == 1
                d = jnp.where(tb, s1 - s0, s0 - s1)
                l = jnp.where(tb, -s1, -s0)
                lt = d < _C07
                asum = asum + jnp.where(lt, l, 0.0)
                aclt = aclt + jnp.where(lt, 1.0, 0.0)
                acle = acle + jnp.where(d <= _C07, 1.0, 0.0)
                return (asum, aclt, acle)

            return lax.fori_loop(0, _SC_CHUNK // 16, vec_body, accs)

        z = jnp.zeros((16,), jnp.float32)
        asum, aclt, acle = lax.fori_loop(0, nch, chunk_body, (z, z, z))
        res_v[pl.ds(0, 16)] = asum
        res_v[pl.ds(16, 16)] = aclt
        res_v[pl.ds(32, 16)] = acle
        res_v[pl.ds(48, 16)] = z
        pltpu.sync_copy(res_v, out_hbm.at[wid])

    return k(score3, target2)


def _ikey(d):
    bits = lax.bitcast_convert_type(d, jnp.int32)
    return jnp.where(bits >= 0, bits,
                     jnp.bitwise_xor(jnp.bitwise_not(bits), jnp.int32(-(2 ** 31))))


def _rare_body(kplus1, nb, score_ref, target_ref, out_ref, st_ref):
    i = pl.program_id(0)
    b = pl.program_id(1)

    @pl.when((i == 0) & (b == 0))
    def _():
        st_ref[0] = jnp.int32(-(2 ** 31))
        st_ref[1] = jnp.int32(2 ** 31 - 1)

    @pl.when((i < 32) & (b == 0))
    def _():
        lo = st_ref[0]
        hi = st_ref[1]
        st_ref[2] = (lo >> 1) + (hi >> 1) + (lo & hi & 1)
        st_ref[3] = jnp.int32(0)

    _, tb, d, l = _per_pixel(score_ref, target_ref)
    key = _ikey(d)

    @pl.when(i < 32)
    def _():
        mid = st_ref[2]
        st_ref[3] = st_ref[3] + jnp.sum((key <= mid).astype(jnp.int32))

    @pl.when((i < 32) & (b == nb - 1))
    def _():
        take_hi = st_ref[3] >= kplus1
        lo = st_ref[0]
        hi = st_ref[1]
        mid = st_ref[2]
        st_ref[0] = jnp.where(take_hi, lo, mid + 1)
        st_ref[1] = jnp.where(take_hi, mid, hi)

    @pl.when(i == 32)
    def _():
        kstar = st_ref[0]
        keep = key < kstar
        ssum = jnp.sum(jnp.where(keep, l, 0.0))
        scnt = jnp.sum(keep.astype(jnp.float32))
        acc = _pack_row_scalars([ssum, scnt])

        @pl.when(b == 0)
        def _():
            out_ref[...] = acc

        @pl.when(b != 0)
        def _():
            out_ref[...] = out_ref[...] + acc


def _rare_ohem(score, target, kplus1):
    B, _, H, W = score.shape
    out = pl.pallas_call(
        lambda sr, tr, orf, st: _rare_body(kplus1, B, sr, tr, orf, st),
        grid=(33, B),
        in_specs=[
            pl.BlockSpec((1, 2, H, W), lambda i, b: (b, 0, 0, 0)),
            pl.BlockSpec((1, H, W), lambda i, b: (b, 0, 0)),
        ],
        out_specs=pl.BlockSpec((8, 128), lambda i, b: (0, 0)),
        out_shape=jax.ShapeDtypeStruct((8, 128), jnp.float32),
        scratch_shapes=[pltpu.SMEM((4,), jnp.int32)],
        compiler_params=pltpu.CompilerParams(
            dimension_semantics=("arbitrary", "arbitrary")),
        interpret=_INTERPRET,
    )(score, target)
    ssum = out[0, 0]
    scnt = out[1, 0]
    return ssum / jnp.maximum(scnt, 1.0)


def kernel(score, target):
    B, C, H, W = score.shape
    target = target.astype(jnp.int32)
    min_kept = int(0.7 * H * W)
    k = min(min_kept, B * H * W - 1)

    out = pl.pallas_call(
        _main_body,
        grid=(B,),
        in_specs=[
            pl.BlockSpec((1, C, H, W), lambda b: (b, 0, 0, 0)),
            pl.BlockSpec((1, H, W), lambda b: (b, 0, 0)),
        ],
        out_specs=pl.BlockSpec((8, 128), lambda b: (0, 0)),
        out_shape=jax.ShapeDtypeStruct((8, 128), jnp.float32),
        scratch_shapes=[
            pltpu.VMEM((H, H), jnp.bfloat16),
            pltpu.VMEM((H, W), jnp.float32),
        ],
        compiler_params=pltpu.CompilerParams(
            dimension_semantics=("arbitrary",)),
        interpret=_INTERPRET,
    )(score, target)

    sum_lt = out[0, 0]
    cnt_lt = out[1, 0]
    edge_sum = out[2, 0]
    edge_cnt = out[3, 0]

    common = cnt_lt >= jnp.float32(k + 1)
    ohem = lax.cond(
        common,
        lambda: sum_lt / jnp.maximum(cnt_lt, 1.0),
        lambda: _rare_ohem(score, target, k + 1),
    )
    edge = edge_sum / jnp.maximum(edge_cnt, 1.0)
    return ohem + 0.5 * edge
